# baseline (speedup 1.0000x reference)
import jax
import jax.numpy as jnp
from jax.experimental import pallas as pl
from jax.experimental.pallas import tpu as pltpu
from jax.experimental.pallas import tpu_sc as plsc

EMB = 128
HID = 512
B = 1024
T = 200
CH = 512

_GATHER_WINDOW = 128
_N_IDX = B * T


def _sc_gather(table, idx):
    n = _N_IDX
    e = table.shape[1]
    mesh = plsc.VectorSubcoreMesh(core_axis_name="core", subcore_axis_name="subcore")

    @pl.kernel(out_type=jax.ShapeDtypeStruct((n, e), table.dtype), mesh=mesh)
    def gather_kernel(tab_hbm, i_hbm, o_hbm):
        def body(i_vmem, o_vmem):
            pltpu.sync_copy(tab_hbm.at[i_vmem.at[0]], o_vmem)

        pltpu.emit_pipeline(
            body,
            grid=(n // _GATHER_WINDOW,),
            in_specs=[pl.BlockSpec((1, _GATHER_WINDOW), index_map=lambda i: (0, i))],
            out_specs=[pl.BlockSpec((_GATHER_WINDOW, e), index_map=lambda i: (i, 0))],
            core_axis_name=("core", "subcore"),
            dimension_semantics=(pltpu.PARALLEL,),
        )(i_hbm, o_hbm)

    return gather_kernel(table, idx)


def _sig(x):
    return 0.5 + 0.5 * jnp.tanh(0.5 * x)


def _gates(g, first, c_ref):
    i = _sig(g[:, :HID])
    f = _sig(g[:, HID:2 * HID])
    gg = jnp.tanh(g[:, 2 * HID:3 * HID])
    o = _sig(g[:, 3 * HID:])
    c_prev = jnp.where(first, jnp.float32(0), c_ref[...])
    c2 = f * c_prev + i * gg
    h2 = o * jnp.tanh(c2)
    c_ref[...] = c2
    return h2.astype(jnp.bfloat16)


def _full_spec(a):
    nd = a.ndim
    return pl.BlockSpec(a.shape, lambda t, _n=nd: (0,) * _n)


def _l0_body(xf_ref, xb_ref, wf_ref, bf_ref, wb_ref, bb_ref,
             hfo_ref, hbo_ref,
             xf0, xf1, cf0, cf1, xb0, xb1, cb0, cb1):
    t = pl.program_id(0)
    first = t == 0

    def chunk(x_ref, w_ref, b_ref, m, xc, c, out_ref):
        @pl.when(first)
        def _():
            xc[:, EMB:] = jnp.zeros_like(xc[:, EMB:])

        xc[:, :EMB] = x_ref[0, m:m + CH].astype(jnp.bfloat16)
        g = jnp.dot(xc[...], w_ref[...], preferred_element_type=jnp.float32)
        g = g + b_ref[...]
        hb = _gates(g, first, c)
        xc[:, EMB:] = hb
        out_ref[0, m:m + CH] = hb

    chunk(xf_ref, wf_ref, bf_ref, 0, xf0, cf0, hfo_ref)
    chunk(xb_ref, wb_ref, bb_ref, 0, xb0, cb0, hbo_ref)
    chunk(xf_ref, wf_ref, bf_ref, CH, xf1, cf1, hfo_ref)
    chunk(xb_ref, wb_ref, bb_ref, CH, xb1, cb1, hbo_ref)


def _bilstm_layer0(emb, wcf, bf, wcb, bb):
    out_shape = [jax.ShapeDtypeStruct((T, B, HID), jnp.bfloat16),
                 jax.ShapeDtypeStruct((T, B, HID), jnp.bfloat16)]
    k0 = EMB + HID
    return pl.pallas_call(
        _l0_body,
        grid=(T,),
        in_specs=[
            pl.BlockSpec((1, B, EMB), lambda t: (t, 0, 0)),
            pl.BlockSpec((1, B, EMB), lambda t: (T - 1 - t, 0, 0)),
            _full_spec(wcf), _full_spec(bf), _full_spec(wcb), _full_spec(bb),
        ],
        out_specs=[
            pl.BlockSpec((1, B, HID), lambda t: (t, 0, 0)),
            pl.BlockSpec((1, B, HID), lambda t: (T - 1 - t, 0, 0)),
        ],
        out_shape=out_shape,
        scratch_shapes=[pltpu.VMEM((CH, k0), jnp.bfloat16),
                        pltpu.VMEM((CH, k0), jnp.bfloat16),
                        pltpu.VMEM((CH, HID), jnp.float32),
                        pltpu.VMEM((CH, HID), jnp.float32),
                        pltpu.VMEM((CH, k0), jnp.bfloat16),
                        pltpu.VMEM((CH, k0), jnp.bfloat16),
                        pltpu.VMEM((CH, HID), jnp.float32),
                        pltpu.VMEM((CH, HID), jnp.float32)],
        compiler_params=pltpu.CompilerParams(dimension_semantics=("arbitrary",)),
    )(emb, emb, wcf, bf, wcb, bb)


def _l1_body(hff_ref, hbf_ref, hfb_ref, hbb_ref,
             wf_ref, bf_ref, wb_ref, bb_ref,
             fwf_ref, fwb_ref, fcb_ref,
             out_ref,
             xf0, xf1, cf0, cf1, xb0, xb1, cb0, cb1):
    t = pl.program_id(0)
    first = t == 0

    def chunk(in1_ref, in2_ref, w_ref, b_ref, m, xc, c):
        @pl.when(first)
        def _():
            xc[:, 2 * HID:] = jnp.zeros_like(xc[:, 2 * HID:])

        xc[:, :HID] = in1_ref[0, m:m + CH]
        xc[:, HID:2 * HID] = in2_ref[0, m:m + CH]
        g = jnp.dot(xc[...], w_ref[...], preferred_element_type=jnp.float32)
        g = g + b_ref[...]
        hb = _gates(g, first, c)
        xc[:, 2 * HID:] = hb

    chunk(hff_ref, hbf_ref, wf_ref, bf_ref, 0, xf0, cf0)
    chunk(hfb_ref, hbb_ref, wb_ref, bb_ref, 0, xb0, cb0)
    chunk(hff_ref, hbf_ref, wf_ref, bf_ref, CH, xf1, cf1)
    chunk(hfb_ref, hbb_ref, wb_ref, bb_ref, CH, xb1, cb1)

    @pl.when(t == T - 1)
    def _():
        for m, xcf, xcb in ((0, xf0, xb0), (CH, xf1, xb1)):
            vf = jnp.sum(xcf[:, 2 * HID:].astype(jnp.float32) * fwf_ref[...],
                         axis=1, keepdims=True)
            vb = jnp.sum(xcb[:, 2 * HID:].astype(jnp.float32) * fwb_ref[...],
                         axis=1, keepdims=True)
            out_ref[m:m + CH] = _sig(vf + vb + fcb_ref[...])


def _bilstm_layer1_fc(hf0, hb0, wcf, bf, wcb, bb, fwf, fwb, fcb):
    seq_spec_f = pl.BlockSpec((1, B, HID), lambda t: (t, 0, 0))
    seq_spec_b = pl.BlockSpec((1, B, HID), lambda t: (T - 1 - t, 0, 0))
    k1 = 3 * HID
    return pl.pallas_call(
        _l1_body,
        grid=(T,),
        in_specs=[
            seq_spec_f, seq_spec_f, seq_spec_b, seq_spec_b,
            _full_spec(wcf), _full_spec(bf), _full_spec(wcb), _full_spec(bb),
            _full_spec(fwf), _full_spec(fwb), _full_spec(fcb),
        ],
        out_specs=pl.BlockSpec((B, 1), lambda t: (0, 0)),
        out_shape=jax.ShapeDtypeStruct((B, 1), jnp.float32),
        scratch_shapes=[pltpu.VMEM((CH, k1), jnp.bfloat16),
                        pltpu.VMEM((CH, k1), jnp.bfloat16),
                        pltpu.VMEM((CH, HID), jnp.float32),
                        pltpu.VMEM((CH, HID), jnp.float32),
                        pltpu.VMEM((CH, k1), jnp.bfloat16),
                        pltpu.VMEM((CH, k1), jnp.bfloat16),
                        pltpu.VMEM((CH, HID), jnp.float32),
                        pltpu.VMEM((CH, HID), jnp.float32)],
        compiler_params=pltpu.CompilerParams(dimension_semantics=("arbitrary",)),
    )(hf0, hb0, hf0, hb0, wcf, bf, wcb, bb, fwf, fwb, fcb)


def kernel(x, table, Wih0f, Whh0f, bih0f, bhh0f, Wih0b, Whh0b, bih0b, bhh0b,
           Wih1f, Whh1f, bih1f, bhh1f, Wih1b, Whh1b, bih1b, bhh1b, fcW, fcb):
    bf16 = jnp.bfloat16

    idx = x.astype(jnp.int32).T.reshape(1, _N_IDX)
    emb = _sc_gather(table, idx).reshape(T, B, EMB)

    wc0f = jnp.concatenate([Wih0f.T, Whh0f.T], axis=0).astype(bf16)
    wc0b = jnp.concatenate([Wih0b.T, Whh0b.T], axis=0).astype(bf16)
    b0f = (bih0f + bhh0f).reshape(1, 4 * HID)
    b0b = (bih0b + bhh0b).reshape(1, 4 * HID)

    hf0, hb0 = _bilstm_layer0(emb, wc0f, b0f, wc0b, b0b)

    wc1f = jnp.concatenate([Wih1f.T, Whh1f.T], axis=0).astype(bf16)
    wc1b = jnp.concatenate([Wih1b.T, Whh1b.T], axis=0).astype(bf16)
    b1f = (bih1f + bhh1f).reshape(1, 4 * HID)
    b1b = (bih1b + bhh1b).reshape(1, 4 * HID)

    fwf = fcW[:, :HID]
    fwb = fcW[:, HID:]
    fcbr = fcb.reshape(1, 1)

    return _bilstm_layer1_fc(hf0, hb0, wc1f, b1f, wc1b, b1b, fwf, fwb, fcbr)


# speedup vs baseline: 1.0552x; 1.0552x over previous
import jax
import jax.numpy as jnp
from jax.experimental import pallas as pl
from jax.experimental.pallas import tpu as pltpu
from jax.experimental.pallas import tpu_sc as plsc

EMB = 128
HID = 512
B = 1024
T = 200

_GATHER_WINDOW = 128
_N_IDX = B * T


def _sc_gather(table, idx):
    n = _N_IDX
    e = table.shape[1]
    mesh = plsc.VectorSubcoreMesh(core_axis_name="core", subcore_axis_name="subcore")

    @pl.kernel(out_type=jax.ShapeDtypeStruct((n, e), table.dtype), mesh=mesh)
    def gather_kernel(tab_hbm, i_hbm, o_hbm):
        def body(i_vmem, o_vmem):
            pltpu.sync_copy(tab_hbm.at[i_vmem.at[0]], o_vmem)

        pltpu.emit_pipeline(
            body,
            grid=(n // _GATHER_WINDOW,),
            in_specs=[pl.BlockSpec((1, _GATHER_WINDOW), index_map=lambda i: (0, i))],
            out_specs=[pl.BlockSpec((_GATHER_WINDOW, e), index_map=lambda i: (i, 0))],
            core_axis_name=("core", "subcore"),
            dimension_semantics=(pltpu.PARALLEL,),
        )(i_hbm, o_hbm)

    return gather_kernel(table, idx)


def _gates(g, first, c_ref):
    half = jnp.bfloat16(0.5)
    i = half + half * jnp.tanh(g[:, :HID])
    f = half + half * jnp.tanh(g[:, HID:2 * HID])
    gg = jnp.tanh(g[:, 2 * HID:3 * HID])
    o = half + half * jnp.tanh(g[:, 3 * HID:])
    c_prev = jnp.where(first, jnp.float32(0), c_ref[...])
    c2 = f.astype(jnp.float32) * c_prev + (i * gg).astype(jnp.float32)
    c_ref[...] = c2
    return o * jnp.tanh(c2).astype(jnp.bfloat16)


def _full_spec(a):
    nd = a.ndim
    return pl.BlockSpec(a.shape, lambda t, _n=nd: (0,) * _n)


def _l0_body(xf_ref, xb_ref, wf_ref, bf_ref, wb_ref, bb_ref,
             hfo_ref, hbo_ref, xcf, cf, xcb, cb):
    t = pl.program_id(0)
    first = t == 0

    def step(x_ref, w_ref, b_ref, xc, c, out_ref):
        @pl.when(first)
        def _():
            xc[:, EMB:] = jnp.zeros_like(xc[:, EMB:])

        xc[:, :EMB] = x_ref[0].astype(jnp.bfloat16)
        g = (jnp.dot(xc[...], w_ref[...], preferred_element_type=jnp.float32)
             ).astype(jnp.bfloat16) + b_ref[...]
        hb = _gates(g, first, c)
        xc[:, EMB:] = hb
        out_ref[0] = hb

    step(xf_ref, wf_ref, bf_ref, xcf, cf, hfo_ref)
    step(xb_ref, wb_ref, bb_ref, xcb, cb, hbo_ref)


def _bilstm_layer0(emb, wcf, bf, wcb, bb):
    out_shape = [jax.ShapeDtypeStruct((T, B, HID), jnp.bfloat16),
                 jax.ShapeDtypeStruct((T, B, HID), jnp.bfloat16)]
    k0 = EMB + HID
    return pl.pallas_call(
        _l0_body,
        grid=(T,),
        in_specs=[
            pl.BlockSpec((1, B, EMB), lambda t: (t, 0, 0)),
            pl.BlockSpec((1, B, EMB), lambda t: (T - 1 - t, 0, 0)),
            _full_spec(wcf), _full_spec(bf), _full_spec(wcb), _full_spec(bb),
        ],
        out_specs=[
            pl.BlockSpec((1, B, HID), lambda t: (t, 0, 0)),
            pl.BlockSpec((1, B, HID), lambda t: (T - 1 - t, 0, 0)),
        ],
        out_shape=out_shape,
        scratch_shapes=[pltpu.VMEM((B, k0), jnp.bfloat16),
                        pltpu.VMEM((B, HID), jnp.float32),
                        pltpu.VMEM((B, k0), jnp.bfloat16),
                        pltpu.VMEM((B, HID), jnp.float32)],
        compiler_params=pltpu.CompilerParams(dimension_semantics=("arbitrary",)),
    )(emb, emb, wcf, bf, wcb, bb)


def _l1_body(hff_ref, hbf_ref, hfb_ref, hbb_ref,
             wf_ref, bf_ref, wb_ref, bb_ref,
             fwf_ref, fwb_ref, fcb_ref,
             out_ref, xcf, cf, xcb, cb):
    t = pl.program_id(0)
    first = t == 0

    def step(in1_ref, in2_ref, w_ref, b_ref, xc, c):
        @pl.when(first)
        def _():
            xc[:, 2 * HID:] = jnp.zeros_like(xc[:, 2 * HID:])

        xc[:, :HID] = in1_ref[0]
        xc[:, HID:2 * HID] = in2_ref[0]
        g = (jnp.dot(xc[...], w_ref[...], preferred_element_type=jnp.float32)
             ).astype(jnp.bfloat16) + b_ref[...]
        hb = _gates(g, first, c)
        xc[:, 2 * HID:] = hb

    step(hff_ref, hbf_ref, wf_ref, bf_ref, xcf, cf)
    step(hfb_ref, hbb_ref, wb_ref, bb_ref, xcb, cb)

    @pl.when(t == T - 1)
    def _():
        vf = jnp.sum(xcf[:, 2 * HID:].astype(jnp.float32) * fwf_ref[...],
                     axis=1, keepdims=True)
        vb = jnp.sum(xcb[:, 2 * HID:].astype(jnp.float32) * fwb_ref[...],
                     axis=1, keepdims=True)
        z = vf + vb + fcb_ref[...]
        out_ref[...] = 0.5 + 0.5 * jnp.tanh(0.5 * z)


def _bilstm_layer1_fc(hf0, hb0, wcf, bf, wcb, bb, fwf, fwb, fcb):
    seq_spec_f = pl.BlockSpec((1, B, HID), lambda t: (t, 0, 0))
    seq_spec_b = pl.BlockSpec((1, B, HID), lambda t: (T - 1 - t, 0, 0))
    k1 = 3 * HID
    return pl.pallas_call(
        _l1_body,
        grid=(T,),
        in_specs=[
            seq_spec_f, seq_spec_f, seq_spec_b, seq_spec_b,
            _full_spec(wcf), _full_spec(bf), _full_spec(wcb), _full_spec(bb),
            _full_spec(fwf), _full_spec(fwb), _full_spec(fcb),
        ],
        out_specs=pl.BlockSpec((B, 1), lambda t: (0, 0)),
        out_shape=jax.ShapeDtypeStruct((B, 1), jnp.float32),
        scratch_shapes=[pltpu.VMEM((B, k1), jnp.bfloat16),
                        pltpu.VMEM((B, HID), jnp.float32),
                        pltpu.VMEM((B, k1), jnp.bfloat16),
                        pltpu.VMEM((B, HID), jnp.float32)],
        compiler_params=pltpu.CompilerParams(dimension_semantics=("arbitrary",)),
    )(hf0, hb0, hf0, hb0, wcf, bf, wcb, bb, fwf, fwb, fcb)


def _prep_w(wih, whh, bih, bhh):
    wc = jnp.concatenate([wih.T, whh.T], axis=0)
    b = (bih + bhh).reshape(1, 4 * HID)
    scale = jnp.concatenate([jnp.full((HID,), 0.5), jnp.full((HID,), 0.5),
                             jnp.ones((HID,)), jnp.full((HID,), 0.5)]).astype(
                                 jnp.float32)
    wc = wc * scale[None, :]
    b = b * scale[None, :]
    return wc.astype(jnp.bfloat16), b.astype(jnp.bfloat16)


def kernel(x, table, Wih0f, Whh0f, bih0f, bhh0f, Wih0b, Whh0b, bih0b, bhh0b,
           Wih1f, Whh1f, bih1f, bhh1f, Wih1b, Whh1b, bih1b, bhh1b, fcW, fcb):
    idx = x.astype(jnp.int32).T.reshape(1, _N_IDX)
    emb = _sc_gather(table, idx).reshape(T, B, EMB)

    wc0f, b0f = _prep_w(Wih0f, Whh0f, bih0f, bhh0f)
    wc0b, b0b = _prep_w(Wih0b, Whh0b, bih0b, bhh0b)
    hf0, hb0 = _bilstm_layer0(emb, wc0f, b0f, wc0b, b0b)

    wc1f, b1f = _prep_w(Wih1f, Whh1f, bih1f, bhh1f)
    wc1b, b1b = _prep_w(Wih1b, Whh1b, bih1b, bhh1b)

    fwf = fcW[:, :HID]
    fwb = fcW[:, HID:]
    fcbr = fcb.reshape(1, 1)

    return _bilstm_layer1_fc(hf0, hb0, wc1f, b1f, wc1b, b1b, fwf, fwb, fcbr)
